# Initial kernel scaffold; baseline (speedup 1.0000x reference)
import jax
import jax.numpy as jnp
from jax.experimental import pallas as pl


def kernel(input_ids, vectors, word_table, pos_table, gamma, beta):
    raise NotImplementedError("write your pallas kernel here")


# speedup vs baseline: 2.7914x; 2.7914x over previous
import functools

import jax
import jax.numpy as jnp
from jax import lax
from jax.experimental import pallas as pl
from jax.experimental.pallas import tpu as pltpu
from jax.experimental.pallas import tpu_sc as plsc

B, L, H = 1024, 200, 128
LP1 = L + 1
NC, NS = 2, 16
NW = NC * NS
NB = B // NW
NL = H // 16
EPS = 1e-12
UNROLL = 3
RSQRT_MAGIC = 0x5F3759DF


def _rsqrt16(a):
    i = plsc.bitcast(a, jnp.int32)
    i = jnp.full((16,), RSQRT_MAGIC, dtype=jnp.int32) - lax.shift_right_logical(i, 1)
    y = plsc.bitcast(i, jnp.float32)
    half_a = a * 0.5
    for _ in range(3):
        y = y * (1.5 - half_a * y * y)
    return y


@functools.partial(
    pl.kernel,
    out_type=jax.ShapeDtypeStruct((B, LP1, H), jnp.float32),
    mesh=plsc.VectorSubcoreMesh(core_axis_name="c", subcore_axis_name="s"),
    compiler_params=pltpu.CompilerParams(
        use_tc_tiling_on_sc=False, needs_layout_passes=False),
    scratch_types=[
        pltpu.VMEM((208,), jnp.int32),
        pltpu.VMEM((LP1, H), jnp.float32),
        pltpu.VMEM((LP1, H), jnp.float32),
        pltpu.VMEM((H,), jnp.float32),
        pltpu.VMEM((H,), jnp.float32),
        pltpu.SemaphoreType.DMA,
    ],
)
def _emb_ln_kernel(ids_hbm, vec_hbm, wt_hbm, pt_hbm, g_hbm, b_hbm,
                   out_hbm, idx_v, rows_v, pos_v, g_v, b_v, gsem):
    wid = lax.axis_index("s") * NC + lax.axis_index("c")

    pltpu.sync_copy(pt_hbm.at[pl.ds(1, LP1)], pos_v)
    pltpu.sync_copy(g_hbm, g_v)
    pltpu.sync_copy(b_hbm, b_v)

    gamma = [g_v[pl.ds(c * 16, 16)] for c in range(NL)]
    beta = [b_v[pl.ds(c * 16, 16)] for c in range(NL)]

    def batch_body(i, carry):
        b = wid * NB + i
        pltpu.sync_copy(ids_hbm.at[b], idx_v.at[pl.ds(0, L)])
        cp0 = pltpu.async_copy(
            wt_hbm.at[idx_v.at[pl.ds(0, 104)]], rows_v.at[pl.ds(0, 104)], gsem)
        cp1 = pltpu.async_copy(
            wt_hbm.at[idx_v.at[pl.ds(104, 96)]], rows_v.at[pl.ds(104, 96)], gsem)
        pltpu.sync_copy(vec_hbm.at[b], rows_v.at[L])
        cp0.wait()
        cp1.wait()

        def row_body(j, carry2):
            for u in range(UNROLL):
                l = j * UNROLL + u
                x = [rows_v[l, pl.ds(c * 16, 16)] + pos_v[l, pl.ds(c * 16, 16)]
                     for c in range(NL)]
                s = (x[0] + x[1]) + (x[2] + x[3]) + ((x[4] + x[5]) + (x[6] + x[7]))
                sq = [xc * xc for xc in x]
                q = (sq[0] + sq[1]) + (sq[2] + sq[3]) + ((sq[4] + sq[5]) + (sq[6] + sq[7]))
                tot = jnp.full((16,), jnp.sum(s))
                tot2 = jnp.full((16,), jnp.sum(q))
                mean = tot * (1.0 / H)
                var = tot2 * (1.0 / H) - mean * mean
                inv = _rsqrt16(var + EPS)
                for c in range(NL):
                    rows_v[l, pl.ds(c * 16, 16)] = (x[c] - mean) * inv * gamma[c] + beta[c]
            return carry2

        lax.fori_loop(0, LP1 // UNROLL, row_body, 0, unroll=False)
        pltpu.sync_copy(rows_v, out_hbm.at[b])
        return carry

    lax.fori_loop(0, NB, batch_body, 0, unroll=False)


def kernel(input_ids, vectors, word_table, pos_table, gamma, beta):
    return _emb_ln_kernel(input_ids.astype(jnp.int32), vectors, word_table,
                          pos_table, gamma, beta)


# speedup vs baseline: 3.4246x; 1.2268x over previous
import functools

import jax
import jax.numpy as jnp
from jax import lax
from jax.experimental import pallas as pl
from jax.experimental.pallas import tpu as pltpu
from jax.experimental.pallas import tpu_sc as plsc

B, L, H = 1024, 200, 128
LP1 = L + 1
NC, NS = 2, 16
NW = NC * NS
NB = B // NW
NL = H // 16
EPS = 1e-12
RU = 4
RSQRT_MAGIC = 0x5F3759DF


def _rsqrt16(a):
    i = plsc.bitcast(a, jnp.int32)
    i = jnp.full((16,), RSQRT_MAGIC, dtype=jnp.int32) - lax.shift_right_logical(i, 1)
    y = plsc.bitcast(i, jnp.float32)
    half_a = a * 0.5
    for _ in range(3):
        y = y * (1.5 - half_a * y * y)
    return y


@functools.partial(
    pl.kernel,
    out_type=jax.ShapeDtypeStruct((B, LP1, H), jnp.float32),
    mesh=plsc.VectorSubcoreMesh(core_axis_name="c", subcore_axis_name="s"),
    compiler_params=pltpu.CompilerParams(
        use_tc_tiling_on_sc=False, needs_layout_passes=False),
    scratch_types=[
        pltpu.VMEM((NB, L), jnp.int32),
        pltpu.VMEM((NB, H), jnp.float32),
        pltpu.VMEM((2, LP1, H), jnp.float32),
        pltpu.VMEM((LP1, H), jnp.float32),
        pltpu.VMEM((H,), jnp.float32),
        pltpu.VMEM((H,), jnp.float32),
        pltpu.SemaphoreType.DMA,
        pltpu.SemaphoreType.DMA,
        pltpu.SemaphoreType.DMA,
        pltpu.SemaphoreType.DMA,
    ],
)
def _emb_ln_kernel(ids_hbm, vec_hbm, wt_hbm, pt_hbm, g_hbm, b_hbm,
                   out_hbm, idx_v, vecs_v, rows_v, pos_v, g_v, b_v,
                   gsem0, gsem1, osem0, osem1):
    wid = lax.axis_index("s") * NC + lax.axis_index("c")
    base = wid * NB
    gsem = (gsem0, gsem1)
    osem = (osem0, osem1)

    pltpu.sync_copy(ids_hbm.at[pl.ds(base, NB)], idx_v)
    pltpu.sync_copy(vec_hbm.at[pl.ds(base, NB)], vecs_v)
    pltpu.sync_copy(pt_hbm.at[pl.ds(1, LP1)], pos_v)
    pltpu.sync_copy(g_hbm, g_v)
    pltpu.sync_copy(b_hbm, b_v)

    gamma = [g_v[pl.ds(c * 16, 16)] for c in range(NL)]
    beta = [b_v[pl.ds(c * 16, 16)] for c in range(NL)]

    def gather_copies(p, i, make_only=False):
        mk = pltpu.make_async_copy if make_only else pltpu.async_copy
        c0 = mk(wt_hbm.at[idx_v.at[i, pl.ds(0, 104)]],
                rows_v.at[p, pl.ds(0, 104)], gsem[p])
        c1 = mk(wt_hbm.at[idx_v.at[i, pl.ds(104, 96)]],
                rows_v.at[p, pl.ds(104, 96)], gsem[p])
        return c0, c1

    def wait_gather(p, i):
        for c in gather_copies(p, i, make_only=True):
            c.wait()

    def ln8(x):
        s = ((x[0] + x[1]) + (x[2] + x[3])) + ((x[4] + x[5]) + (x[6] + x[7]))
        sq = [xc * xc for xc in x]
        q = ((sq[0] + sq[1]) + (sq[2] + sq[3])) + ((sq[4] + sq[5]) + (sq[6] + sq[7]))
        tot = jnp.full((16,), jnp.sum(s))
        tot2 = jnp.full((16,), jnp.sum(q))
        mean = tot * (1.0 / H)
        var = tot2 * (1.0 / H) - mean * mean
        inv = _rsqrt16(var + EPS)
        return [(x[c] - mean) * inv * gamma[c] + beta[c] for c in range(NL)]

    def compute(p, i):
        def row_body(j, c2):
            for u in range(RU):
                l = j * RU + u
                x = [rows_v[p, l, pl.ds(c * 16, 16)] + pos_v[l, pl.ds(c * 16, 16)]
                     for c in range(NL)]
                o = ln8(x)
                for c in range(NL):
                    rows_v[p, l, pl.ds(c * 16, 16)] = o[c]
            return c2

        lax.fori_loop(0, L // RU, row_body, 0, unroll=False)
        x = [vecs_v[i, pl.ds(c * 16, 16)] + pos_v[L, pl.ds(c * 16, 16)]
             for c in range(NL)]
        o = ln8(x)
        for c in range(NL):
            rows_v[p, L, pl.ds(c * 16, 16)] = o[c]

    gather_copies(0, 0)

    def pair_body(j, carry):
        i0 = 2 * j
        i1 = i0 + 1

        @pl.when(j > 0)
        def _():
            pltpu.make_async_copy(rows_v.at[1], out_hbm.at[base + i0 - 1],
                                  osem[1]).wait()

        gather_copies(1, i1)
        wait_gather(0, i0)
        compute(0, i0)
        out0 = pltpu.async_copy(rows_v.at[0], out_hbm.at[base + i0], osem[0])
        wait_gather(1, i1)
        out0.wait()

        @pl.when(j < NB // 2 - 1)
        def _():
            gather_copies(0, i0 + 2)
        compute(1, i1)
        pltpu.async_copy(rows_v.at[1], out_hbm.at[base + i1], osem[1])
        return carry

    lax.fori_loop(0, NB // 2, pair_body, 0, unroll=False)
    pltpu.make_async_copy(rows_v.at[1], out_hbm.at[base + NB - 1],
                          osem[1]).wait()


def kernel(input_ids, vectors, word_table, pos_table, gamma, beta):
    return _emb_ln_kernel(input_ids.astype(jnp.int32), vectors, word_table,
                          pos_table, gamma, beta)


# speedup vs baseline: 4.3553x; 1.2718x over previous
import functools

import jax
import jax.numpy as jnp
from jax import lax
from jax.experimental import pallas as pl
from jax.experimental.pallas import tpu as pltpu
from jax.experimental.pallas import tpu_sc as plsc

B, L, H = 1024, 200, 128
LP1 = L + 1
LPAD = 208
NC, NS = 2, 16
NW = NC * NS
NB = B // NW
NL = H // 16
EPS = 1e-12
RU = 4
RSQRT_MAGIC = 0x5F3759DF


def _rsqrt16(a):
    i = plsc.bitcast(a, jnp.int32)
    i = jnp.full((16,), RSQRT_MAGIC, dtype=jnp.int32) - lax.shift_right_logical(i, 1)
    y = plsc.bitcast(i, jnp.float32)
    half_a = a * 0.5
    for _ in range(3):
        y = y * (1.5 - half_a * y * y)
    return y


@functools.partial(
    pl.kernel,
    out_type=jax.ShapeDtypeStruct((B, LPAD, H), jnp.float32),
    mesh=plsc.VectorSubcoreMesh(core_axis_name="c", subcore_axis_name="s"),
    compiler_params=pltpu.CompilerParams(
        use_tc_tiling_on_sc=False, needs_layout_passes=False),
    scratch_types=[
        pltpu.VMEM((NB, L), jnp.int32),
        pltpu.VMEM((NB, H), jnp.float32),
        pltpu.VMEM((2, LP1, H), jnp.float32),
        pltpu.VMEM((LP1, H), jnp.float32),
        pltpu.VMEM((H,), jnp.float32),
        pltpu.VMEM((H,), jnp.float32),
        pltpu.SemaphoreType.DMA,
        pltpu.SemaphoreType.DMA,
        pltpu.SemaphoreType.DMA,
        pltpu.SemaphoreType.DMA,
    ],
)
def _emb_ln_kernel(ids_hbm, vec_hbm, wt_hbm, pt_hbm, g_hbm, b_hbm,
                   out_hbm, idx_v, vecs_v, rows_v, pos_v, g_v, b_v,
                   gsem0, gsem1, osem0, osem1):
    wid = lax.axis_index("s") * NC + lax.axis_index("c")
    base = wid * NB
    gsem = (gsem0, gsem1)
    osem = (osem0, osem1)

    pltpu.sync_copy(ids_hbm.at[pl.ds(base, NB)], idx_v)
    pltpu.sync_copy(vec_hbm.at[pl.ds(base, NB)], vecs_v)
    pltpu.sync_copy(pt_hbm.at[pl.ds(1, LP1)], pos_v)
    pltpu.sync_copy(g_hbm, g_v)
    pltpu.sync_copy(b_hbm, b_v)

    gamma = [g_v[pl.ds(c * 16, 16)] for c in range(NL)]
    beta = [b_v[pl.ds(c * 16, 16)] for c in range(NL)]

    def gather_copies(p, i, make_only=False):
        mk = pltpu.make_async_copy if make_only else pltpu.async_copy
        c0 = mk(wt_hbm.at[idx_v.at[i, pl.ds(0, 104)]],
                rows_v.at[p, pl.ds(0, 104)], gsem[p])
        c1 = mk(wt_hbm.at[idx_v.at[i, pl.ds(104, 96)]],
                rows_v.at[p, pl.ds(104, 96)], gsem[p])
        return c0, c1

    def wait_gather(p, i):
        for c in gather_copies(p, i, make_only=True):
            c.wait()

    def ln8(x):
        s = ((x[0] + x[1]) + (x[2] + x[3])) + ((x[4] + x[5]) + (x[6] + x[7]))
        sq = [xc * xc for xc in x]
        q = ((sq[0] + sq[1]) + (sq[2] + sq[3])) + ((sq[4] + sq[5]) + (sq[6] + sq[7]))
        tot = jnp.full((16,), jnp.sum(s))
        tot2 = jnp.full((16,), jnp.sum(q))
        mean = tot * (1.0 / H)
        var = tot2 * (1.0 / H) - mean * mean
        inv = _rsqrt16(var + EPS)
        return [(x[c] - mean) * inv * gamma[c] + beta[c] for c in range(NL)]

    def compute(p, i):
        def row_body(j, c2):
            for u in range(RU):
                l = j * RU + u
                x = [rows_v[p, l, pl.ds(c * 16, 16)] + pos_v[l, pl.ds(c * 16, 16)]
                     for c in range(NL)]
                o = ln8(x)
                for c in range(NL):
                    rows_v[p, l, pl.ds(c * 16, 16)] = o[c]
            return c2

        lax.fori_loop(0, L // RU, row_body, 0, unroll=False)
        x = [vecs_v[i, pl.ds(c * 16, 16)] + pos_v[L, pl.ds(c * 16, 16)]
             for c in range(NL)]
        o = ln8(x)
        for c in range(NL):
            rows_v[p, L, pl.ds(c * 16, 16)] = o[c]

    gather_copies(0, 0)

    def pair_body(j, carry):
        i0 = 2 * j
        i1 = i0 + 1

        @pl.when(j > 0)
        def _():
            pltpu.make_async_copy(rows_v.at[1], out_hbm.at[base + i0 - 1, pl.ds(0, LP1)],
                                  osem[1]).wait()

        gather_copies(1, i1)
        wait_gather(0, i0)
        compute(0, i0)
        out0 = pltpu.async_copy(rows_v.at[0], out_hbm.at[base + i0, pl.ds(0, LP1)], osem[0])
        wait_gather(1, i1)
        out0.wait()

        @pl.when(j < NB // 2 - 1)
        def _():
            gather_copies(0, i0 + 2)
        compute(1, i1)
        pltpu.async_copy(rows_v.at[1], out_hbm.at[base + i1, pl.ds(0, LP1)], osem[1])
        return carry

    lax.fori_loop(0, NB // 2, pair_body, 0, unroll=False)
    pltpu.make_async_copy(rows_v.at[1], out_hbm.at[base + NB - 1, pl.ds(0, LP1)],
                          osem[1]).wait()


def kernel(input_ids, vectors, word_table, pos_table, gamma, beta):
    out = _emb_ln_kernel(input_ids.astype(jnp.int32), vectors, word_table,
                         pos_table, gamma, beta)
    return out[:, :LP1, :]
